# baseline (speedup 1.0000x reference)
import numpy as np
import jax
import jax.numpy as jnp
from jax import lax
from jax.experimental import pallas as pl
from jax.experimental.pallas import tpu as pltpu

_F32 = jnp.float32
_NT = (((1,), (1,)), ((), ()))
_TN = (((0,), (0,)), ((), ()))


def _row_softmax(l):
    m = jnp.max(l, axis=-1, keepdims=True)
    p = jnp.exp(l - m)
    return p / jnp.sum(p, axis=-1, keepdims=True)


def _leaky(x):
    return jnp.where(x >= 0, x, 0.05 * x)


def _first_stage(xv4, xv2, W, b):
    B, V, T, C = xv4.shape
    O = W.shape[0]

    def body(x4_ref, x2_ref, w_ref, brow_ref, h_ref, a_ref):
        e = jnp.mean(x4_ref[0], axis=1)
        l = lax.dot_general(e, e, _NT, preferred_element_type=_F32)
        a_ref[0] = _row_softmax(l * (1.0 / np.sqrt(C)))
        h = lax.dot_general(x2_ref[0], w_ref[...], _NT,
                            preferred_element_type=_F32)
        h_ref[0] = h + brow_ref[...]

    return pl.pallas_call(
        body,
        grid=(B,),
        in_specs=[
            pl.BlockSpec((1, V, T, C), lambda b: (b, 0, 0, 0)),
            pl.BlockSpec((1, V * T, C), lambda b: (b, 0, 0)),
            pl.BlockSpec((O, C), lambda b: (0, 0)),
            pl.BlockSpec((1, O), lambda b: (0, 0)),
        ],
        out_specs=[
            pl.BlockSpec((1, V * T, O), lambda b: (b, 0, 0)),
            pl.BlockSpec((1, V, V), lambda b: (b, 0, 0)),
        ],
        out_shape=[
            jax.ShapeDtypeStruct((B, V * T, O), _F32),
            jax.ShapeDtypeStruct((B, V, V), _F32),
        ],
    )(xv4, xv2, W, b.reshape(1, O))


def _amix(H, A):
    B, V, N = H.shape

    def body(h_ref, a_ref, y_ref):
        y_ref[0] = lax.dot_general(a_ref[0], h_ref[0], _TN,
                                   preferred_element_type=_F32)

    return pl.pallas_call(
        body,
        grid=(B,),
        in_specs=[
            pl.BlockSpec((1, V, N), lambda b: (b, 0, 0)),
            pl.BlockSpec((1, V, V), lambda b: (b, 0, 0)),
        ],
        out_specs=pl.BlockSpec((1, V, N), lambda b: (b, 0, 0)),
        out_shape=jax.ShapeDtypeStruct((B, V, N), _F32),
    )(H, A)


def _mid_stage(Y, g, be, W, b, T, V):
    B = Y.shape[0]
    C = Y.shape[2] // T
    O = W.shape[0]
    Y2 = Y.reshape(B, V * T, C)
    Y4 = Y.reshape(B, V, T, C)
    n = B * T * V

    def body(y2_ref, y4_ref, g_ref, be_ref, w_ref, brow_ref, h_ref, a_ref,
             acc):
        ph = pl.program_id(0)
        b_i = pl.program_id(1)

        @pl.when((ph == 0) & (b_i == 0))
        def _():
            acc[...] = jnp.zeros((8, C), _F32)

        @pl.when(ph == 0)
        def _():
            y = y2_ref[0]
            acc[0:1, :] += jnp.sum(y, axis=0, keepdims=True)
            acc[1:2, :] += jnp.sum(y * y, axis=0, keepdims=True)

        @pl.when(ph == 1)
        def _():
            mean = acc[0:1, :] * (1.0 / n)
            var = acc[1:2, :] * (1.0 / n) - mean * mean
            inv = lax.rsqrt(var + 1e-5)
            scale = g_ref[...] * inv
            shift = be_ref[...] - mean * scale
            z2 = _leaky(y2_ref[0] * scale + shift)
            h = lax.dot_general(z2, w_ref[...], _NT,
                                preferred_element_type=_F32)
            h_ref[0] = h + brow_ref[...]
            z4 = _leaky(y4_ref[0] * scale[None] + shift[None])
            e = jnp.mean(z4, axis=1)
            l = lax.dot_general(e, e, _NT, preferred_element_type=_F32)
            a_ref[0] = _row_softmax(l * (1.0 / np.sqrt(C)))

    return pl.pallas_call(
        body,
        grid=(2, B),
        in_specs=[
            pl.BlockSpec((1, V * T, C), lambda ph, b: (b, 0, 0)),
            pl.BlockSpec((1, V, T, C), lambda ph, b: (b, 0, 0, 0)),
            pl.BlockSpec((1, C), lambda ph, b: (0, 0)),
            pl.BlockSpec((1, C), lambda ph, b: (0, 0)),
            pl.BlockSpec((O, C), lambda ph, b: (0, 0)),
            pl.BlockSpec((1, O), lambda ph, b: (0, 0)),
        ],
        out_specs=[
            pl.BlockSpec((1, V * T, O), lambda ph, b: (b, 0, 0)),
            pl.BlockSpec((1, V, V), lambda ph, b: (b, 0, 0)),
        ],
        out_shape=[
            jax.ShapeDtypeStruct((B, V * T, O), _F32),
            jax.ShapeDtypeStruct((B, V, V), _F32),
        ],
        scratch_shapes=[pltpu.VMEM((8, C), _F32)],
    )(Y2, Y4, g.reshape(1, C), be.reshape(1, C), W, b.reshape(1, O))


def kernel(x, W1, b1, g1, be1, W2, b2, g2, be2, W3, b3):
    B, C0, T, V = x.shape
    O1, O2, O3 = W1.shape[0], W2.shape[0], W3.shape[0]
    xv4 = jnp.transpose(x, (0, 3, 2, 1))
    xv2 = xv4.reshape(B, V * T, C0)
    H1, A1 = _first_stage(xv4, xv2, W1, b1)
    Y1 = _amix(H1.reshape(B, V, T * O1), A1)
    H2, A2 = _mid_stage(Y1, g1, be1, W2, b2, T, V)
    Y2 = _amix(H2.reshape(B, V, T * O2), A2)
    H3, A3 = _mid_stage(Y2, g2, be2, W3, b3, T, V)
    Y3 = _amix(H3.reshape(B, V, T * O3), A3)
    return Y3.reshape(B, V, T, O3).transpose(0, 3, 2, 1)
